# baseline (speedup 1.0000x reference)
import functools

import jax
import jax.numpy as jnp
from jax import lax
from jax.experimental import pallas as pl
from jax.experimental.pallas import tpu as pltpu
from jax.experimental.pallas import tpu_sc as plsc

B = 16384
D = 256
CB = 3
NC = 2
NS = 16
NW = NC * NS
BPW = B // NW
CHUNK = 64
NCH = BPW // CHUNK
LANES = 16
DJ = D // LANES


def _body(item_hbm, cl0, cl1, cl2, w0, w1, w2, out_hbm,
          item_v, codes_v, emb_v, out_v, sem):
    wid = lax.axis_index("s") * NC + lax.axis_index("c")
    cls = (cl0, cl1, cl2)
    ws = (w0, w1, w2)
    iota = lax.iota(jnp.int32, LANES)

    def chunk_body(c, _):
        row0 = wid * BPW + c * CHUNK
        pltpu.sync_copy(item_hbm.at[wid, c], item_v)
        for i in range(CB):
            pltpu.async_copy(cls[i].at[item_v], codes_v, sem).wait()
            pltpu.async_copy(ws[i].at[codes_v], emb_v, sem).wait()

            def r_body(r, _):
                for j in range(DJ):
                    v = emb_v[r, pl.ds(j * LANES, LANES)]
                    cols = 3 * iota + (r * (D * CB) + 3 * j * LANES + i)
                    plsc.store_scatter(out_v, [cols], v)
                return _

            lax.fori_loop(0, CHUNK, r_body, None)
        pltpu.sync_copy(out_v, out_hbm.at[pl.ds(row0 * (D * CB), CHUNK * D * CB)])
        return _

    lax.fori_loop(0, NCH, chunk_body, None)


@functools.partial(jax.jit, static_argnames=())
def _rpq(item3, cl0, cl1, cl2, w0, w1, w2):
    mesh = plsc.VectorSubcoreMesh(
        core_axis_name="c", subcore_axis_name="s",
        num_cores=NC, num_subcores=NS)
    return pl.kernel(
        _body,
        out_type=jax.ShapeDtypeStruct((B * D * CB,), jnp.float32),
        mesh=mesh,
        scratch_types=[
            pltpu.VMEM((CHUNK,), jnp.int32),
            pltpu.VMEM((CHUNK,), jnp.int32),
            pltpu.VMEM((CHUNK, D), jnp.float32),
            pltpu.VMEM((CHUNK * D * CB,), jnp.float32),
            pltpu.SemaphoreType.DMA,
        ],
        compiler_params=pltpu.CompilerParams(needs_layout_passes=False),
    )(item3, cl0, cl1, cl2, w0, w1, w2)


def kernel(item, code_list_0, code_list_1, code_list_2, W0, W1, W2):
    item3 = item.reshape(NW, NCH, CHUNK)
    out = _rpq(item3, code_list_0, code_list_1, code_list_2, W0, W1, W2)
    return out.reshape(B, D, CB)


# speedup vs baseline: 1.0127x; 1.0127x over previous
import functools

import jax
import jax.numpy as jnp
from jax import lax
from jax.experimental import pallas as pl
from jax.experimental.pallas import tpu as pltpu
from jax.experimental.pallas import tpu_sc as plsc

B = 16384
D = 256
CB = 3
NC = 2
NS = 16
NW = NC * NS
BPW = B // NW
CHUNK = 64
NCH = BPW // CHUNK
LANES = 16
DJ = D // LANES
NBUF = 4
OROW = D * CB


def _body(item_hbm, cl0, cl1, cl2, w0, w1, w2, out_hbm,
          item_v, codes_v, embs_and_sems):
    (emb0, emb1, emb2, emb3, out_v,
     sem_c, sem_o, sem0, sem1, sem2, sem3) = embs_and_sems
    embufs = (emb0, emb1, emb2, emb3)
    sems = (sem0, sem1, sem2, sem3)
    wid = lax.axis_index("s") * NC + lax.axis_index("c")
    cls = (cl0, cl1, cl2)
    ws = (w0, w1, w2)
    colbase = 3 * lax.iota(jnp.int32, LANES)

    pltpu.async_copy(item_hbm.at[wid], item_v, sem_c).wait()

    cdescs = []
    for c in range(NCH):
        for i in range(CB):
            cdescs.append(pltpu.async_copy(
                cls[i].at[item_v.at[c]], codes_v.at[i, c], sem_c))
    for d in cdescs:
        d.wait()

    steps = [(c, i) for c in range(NCH) for i in range(CB)]

    def fire(s):
        c, i = steps[s]
        b = s % NBUF
        return pltpu.async_copy(ws[i].at[codes_v.at[i, c]], embufs[b], sems[b])

    def interleave(s):
        c, i = steps[s]
        emb = embufs[s % NBUF]

        def r_body(r, _):
            base = r * OROW + i
            for j in range(DJ):
                v = emb[r, pl.ds(j * LANES, LANES)]
                plsc.store_scatter(out_v, [colbase + (base + 3 * j * LANES)], v)
            return _

        lax.fori_loop(0, CHUNK, r_body, None, unroll=2)

    descs = {}
    for s in range(min(NBUF, len(steps))):
        descs[s] = fire(s)
    out_desc = None
    for s, (c, i) in enumerate(steps):
        descs[s].wait()
        if i == 0 and out_desc is not None:
            out_desc.wait()
        interleave(s)
        if s + NBUF < len(steps):
            descs[s + NBUF] = fire(s + NBUF)
        if i == CB - 1:
            row0 = wid * BPW + c * CHUNK
            out_desc = pltpu.async_copy(
                out_v, out_hbm.at[pl.ds(row0 * OROW, CHUNK * OROW)], sem_o)
    out_desc.wait()


@jax.jit
def _rpq(item3, cl0, cl1, cl2, w0, w1, w2):
    mesh = plsc.VectorSubcoreMesh(
        core_axis_name="c", subcore_axis_name="s",
        num_cores=NC, num_subcores=NS)
    return pl.kernel(
        _body,
        out_type=jax.ShapeDtypeStruct((B * OROW,), jnp.float32),
        mesh=mesh,
        scratch_types=[
            pltpu.VMEM((NCH, CHUNK), jnp.int32),
            pltpu.VMEM((CB, NCH, CHUNK), jnp.int32),
            [pltpu.VMEM((CHUNK, D), jnp.float32)] * NBUF
            + [pltpu.VMEM((CHUNK * OROW,), jnp.float32)]
            + [pltpu.SemaphoreType.DMA] * (2 + NBUF),
        ],
        compiler_params=pltpu.CompilerParams(needs_layout_passes=False),
    )(item3, cl0, cl1, cl2, w0, w1, w2)


def kernel(item, code_list_0, code_list_1, code_list_2, W0, W1, W2):
    item3 = item.reshape(NW, NCH, CHUNK)
    out = _rpq(item3, code_list_0, code_list_1, code_list_2, W0, W1, W2)
    return out.reshape(B, D, CB)
